# baseline (speedup 1.0000x reference)
import functools

import jax
import jax.numpy as jnp
from jax import lax
from jax.experimental import pallas as pl
from jax.experimental.pallas import tpu as pltpu
from jax.experimental.pallas import tpu_sc as plsc

B = 16384
D = 32

_info = plsc.get_sparse_core_info()
_NC, _NS = _info.num_cores, _info.num_subcores
_NW = _NC * _NS
_BPW = B // _NW
_CH = _BPW // 2


def _sc_gather(uid, iid, ug_t, ig_t, um_t, im_t):
    mesh = plsc.VectorSubcoreMesh(core_axis_name="c", subcore_axis_name="s")
    out_t = [jax.ShapeDtypeStruct((D, B), jnp.float32)] * 4

    @functools.partial(
        pl.kernel,
        mesh=mesh,
        out_type=out_t,
        compiler_params=pltpu.CompilerParams(use_tc_tiling_on_sc=False),
        scratch_types=[
            pltpu.VMEM((_BPW,), jnp.int32),
            pltpu.VMEM((_BPW,), jnp.int32),
            pltpu.VMEM((D, _CH), jnp.float32),
            pltpu.VMEM((D, _CH), jnp.float32),
        ],
    )
    def k(uid_h, iid_h, ugt_h, igt_h, umt_h, imt_h,
          oug_h, oig_h, oum_h, oim_h,
          idx_u, idx_i, b0, b1):
        wid = lax.axis_index("s") * _NC + lax.axis_index("c")
        base = wid * _BPW
        pltpu.sync_copy(uid_h.at[pl.ds(base, _BPW)], idx_u)
        pltpu.sync_copy(iid_h.at[pl.ds(base, _BPW)], idx_i)

        rounds = []
        for tbl, idx, out in ((ugt_h, idx_u, oug_h), (igt_h, idx_i, oig_h),
                              (umt_h, idx_u, oum_h), (imt_h, idx_i, oim_h)):
            for c in range(2):
                rounds.append((tbl, idx, out, c))
        bufs = (b0, b1)

        def body(g0, g1, s0, s1):
            gsems = (g0, g1)
            ssems = (s0, s1)
            stores = [None, None]
            for t, (tbl, idx, out, c) in enumerate(rounds):
                bi = t % 2
                buf = bufs[bi]
                if stores[bi] is not None:
                    stores[bi].wait()
                sub = idx.at[pl.ds(c * _CH, _CH)]
                grp = []
                for f in range(D):
                    grp.append(pltpu.async_copy(
                        tbl.at[f].at[sub], buf.at[f], gsems[bi]))
                for d in grp:
                    d.wait()
                stores[bi] = pltpu.async_copy(
                    buf, out.at[:, pl.ds(base + c * _CH, _CH)], ssems[bi])
            for s in stores:
                s.wait()

        pl.run_scoped(body, pltpu.SemaphoreType.DMA(()),
                      pltpu.SemaphoreType.DMA(()),
                      pltpu.SemaphoreType.DMA(()),
                      pltpu.SemaphoreType.DMA(()))

    return k(uid, iid, ug_t, ig_t, um_t, im_t)


_BLK = 2048
_FEAT_DOT = (((0,), (0,)), ((), ()))


def _mlp_body(ug, ig, um, im, w1a, w1b, b1, w2, b2, w3, b3, wpg, wph, bp, out):
    h = jnp.maximum(
        lax.dot_general(um[...], w1a[...], _FEAT_DOT,
                        preferred_element_type=jnp.float32)
        + lax.dot_general(im[...], w1b[...], _FEAT_DOT,
                          preferred_element_type=jnp.float32)
        + b1[...], 0.0)
    h = jnp.maximum(
        jnp.dot(h, w2[...], preferred_element_type=jnp.float32) + b2[...], 0.0)
    h = jnp.maximum(
        jnp.dot(h, w3[...], preferred_element_type=jnp.float32) + b3[...], 0.0)
    g = ug[...] * ig[...]
    logit = (jnp.sum(g * wpg[...], axis=0)
             + jnp.sum(h * wph[...], axis=1) + bp[0, 0])
    out[...] = jax.nn.sigmoid(logit)


def _tc_mlp(ugt, igt, umt, imt, W1, b1, W2, b2, W3, b3, Wp, bp):
    w1a, w1b = W1[:D], W1[D:]
    wpg = Wp[:D, 0].reshape(D, 1)
    wph = Wp[D:, 0].reshape(1, D)
    b1r = b1.reshape(1, -1)
    b2r = b2.reshape(1, -1)
    b3r = b3.reshape(1, -1)
    bpr = bp.reshape(1, 1)

    grid = B // _BLK
    col_spec = pl.BlockSpec((D, _BLK), lambda i: (0, i))
    full = lambda a: pl.BlockSpec(a.shape, lambda i: (0,) * a.ndim)
    return pl.pallas_call(
        _mlp_body,
        grid=(grid,),
        in_specs=[
            col_spec, col_spec, col_spec, col_spec,
            full(w1a), full(w1b), full(b1r),
            full(W2), full(b2r), full(W3), full(b3r),
            full(wpg), full(wph),
            pl.BlockSpec(memory_space=pltpu.SMEM),
        ],
        out_specs=pl.BlockSpec((_BLK,), lambda i: (i,)),
        out_shape=jax.ShapeDtypeStruct((B,), jnp.float32),
    )(ugt, igt, umt, imt, w1a, w1b, b1r, W2, b2r, W3, b3r, wpg, wph, bpr)


def kernel(user_ids, item_ids, user_emb_gmf, item_emb_gmf, user_emb_mlp,
           item_emb_mlp, W1, b1, W2, b2, W3, b3, Wp, bp):
    ugw, igw, umw, imw = _sc_gather(user_ids, item_ids, user_emb_gmf.T,
                                    item_emb_gmf.T, user_emb_mlp.T,
                                    item_emb_mlp.T)
    return _tc_mlp(ugw, igw, umw, imw, W1, b1, W2, b2, W3, b3, Wp, bp)


# speedup vs baseline: 7.9734x; 7.9734x over previous
import functools

import jax
import jax.numpy as jnp
from jax import lax
from jax.experimental import pallas as pl
from jax.experimental.pallas import tpu as pltpu
from jax.experimental.pallas import tpu_sc as plsc

B = 16384
D = 32

_info = plsc.get_sparse_core_info()
_NC, _NS = _info.num_cores, _info.num_subcores
_NW = _NC * _NS
_BPW = B // _NW
_CH = _BPW // 2


def _sc_gather(uid, iid, ug_t, ig_t, um_t, im_t):
    mesh = plsc.VectorSubcoreMesh(core_axis_name="c", subcore_axis_name="s")
    out_t = [jax.ShapeDtypeStruct((B, D), jnp.float32)] * 4

    @functools.partial(
        pl.kernel,
        mesh=mesh,
        out_type=out_t,
        scratch_types=[
            pltpu.VMEM((_BPW,), jnp.int32),
            pltpu.VMEM((_BPW,), jnp.int32),
            pltpu.VMEM((_CH, D), jnp.float32),
            pltpu.VMEM((_CH, D), jnp.float32),
        ],
    )
    def k(uid_h, iid_h, ugt_h, igt_h, umt_h, imt_h,
          oug_h, oig_h, oum_h, oim_h,
          idx_u, idx_i, b0, b1):
        wid = lax.axis_index("s") * _NC + lax.axis_index("c")
        base = wid * _BPW
        pltpu.sync_copy(uid_h.at[pl.ds(base, _BPW)], idx_u)
        pltpu.sync_copy(iid_h.at[pl.ds(base, _BPW)], idx_i)

        rounds = []
        for tbl, idx, out in ((ugt_h, idx_u, oug_h), (igt_h, idx_i, oig_h),
                              (umt_h, idx_u, oum_h), (imt_h, idx_i, oim_h)):
            for c in range(2):
                rounds.append((tbl, idx, out, c))
        bufs = (b0, b1)

        def body(g0, g1, s0, s1):
            gsems = (g0, g1)
            ssems = (s0, s1)
            stores = [None, None]
            ngrp = _CH // 16
            for t, (tbl, idx, out, c) in enumerate(rounds):
                bi = t % 2
                buf = bufs[bi]
                off = c * _CH
                if stores[bi] is not None:
                    stores[bi].wait()

                def fire(g, _):
                    vec = idx[pl.ds(off + g * 16, 16)]
                    grp = []
                    for kk in range(16):
                        grp.append(pltpu.async_copy(
                            tbl.at[pl.ds(vec[kk], 1), :],
                            buf.at[pl.ds(g * 16 + kk, 1), :],
                            gsems[bi]))
                    for d in grp:
                        d.wait()
                    return ()

                lax.fori_loop(0, ngrp, fire, ())
                stores[bi] = pltpu.async_copy(
                    buf, out.at[pl.ds(base + off, _CH)], ssems[bi])
            for s in stores:
                s.wait()

        pl.run_scoped(body, pltpu.SemaphoreType.DMA(()),
                      pltpu.SemaphoreType.DMA(()),
                      pltpu.SemaphoreType.DMA(()),
                      pltpu.SemaphoreType.DMA(()))

    return k(uid, iid, ug_t, ig_t, um_t, im_t)


_BLK = 2048


def _mlp_body(ug, ig, um, im, w1a, w1b, b1, w2, b2, w3, b3, wpg, wph, bp, out):
    h = jnp.maximum(
        jnp.dot(um[...], w1a[...], preferred_element_type=jnp.float32)
        + jnp.dot(im[...], w1b[...], preferred_element_type=jnp.float32)
        + b1[...], 0.0)
    h = jnp.maximum(
        jnp.dot(h, w2[...], preferred_element_type=jnp.float32) + b2[...], 0.0)
    h = jnp.maximum(
        jnp.dot(h, w3[...], preferred_element_type=jnp.float32) + b3[...], 0.0)
    g = ug[...] * ig[...]
    logit = (jnp.sum(g * wpg[...], axis=1)
             + jnp.sum(h * wph[...], axis=1) + bp[0, 0])
    out[...] = jax.nn.sigmoid(logit)


def _tc_mlp(ug, ig, um, im, W1, b1, W2, b2, W3, b3, Wp, bp):
    w1a, w1b = W1[:D], W1[D:]
    wpg = Wp[:D, 0].reshape(1, D)
    wph = Wp[D:, 0].reshape(1, D)
    b1r = b1.reshape(1, -1)
    b2r = b2.reshape(1, -1)
    b3r = b3.reshape(1, -1)
    bpr = bp.reshape(1, 1)

    grid = B // _BLK
    row_spec = pl.BlockSpec((_BLK, D), lambda i: (i, 0))
    full = lambda a: pl.BlockSpec(a.shape, lambda i: (0,) * a.ndim)
    return pl.pallas_call(
        _mlp_body,
        grid=(grid,),
        in_specs=[
            row_spec, row_spec, row_spec, row_spec,
            full(w1a), full(w1b), full(b1r),
            full(W2), full(b2r), full(W3), full(b3r),
            full(wpg), full(wph),
            pl.BlockSpec(memory_space=pltpu.SMEM),
        ],
        out_specs=pl.BlockSpec((_BLK,), lambda i: (i,)),
        out_shape=jax.ShapeDtypeStruct((B,), jnp.float32),
    )(ug, ig, um, im, w1a, w1b, b1r, W2, b2r, W3, b3r, wpg, wph, bpr)


def kernel(user_ids, item_ids, user_emb_gmf, item_emb_gmf, user_emb_mlp,
           item_emb_mlp, W1, b1, W2, b2, W3, b3, Wp, bp):
    ug, ig, um, im = _sc_gather(user_ids, item_ids, user_emb_gmf,
                                item_emb_gmf, user_emb_mlp, item_emb_mlp)
    return _tc_mlp(ug, ig, um, im, W1, b1, W2, b2, W3, b3, Wp, bp)
